# baseline (speedup 1.0000x reference)
import functools

import jax
import jax.numpy as jnp
from jax import lax
from jax.experimental import pallas as pl
from jax.experimental.pallas import tpu as pltpu
from jax.experimental.pallas import tpu_sc as plsc

V = 100000
D = 128
B = 16384
K = 5
L = 16
NC = 2
NS = 16
NW = NC * NS
BPW = B // NW
C = 32
G = BPW // C
NV = D // L


def _sc_scores(pos_v, pos_u, neg_u, v_weight, u_weight):
    mesh = plsc.VectorSubcoreMesh(core_axis_name="c", subcore_axis_name="s")

    @functools.partial(
        pl.kernel,
        mesh=mesh,
        out_type=[
            jax.ShapeDtypeStruct((B,), jnp.float32),
            jax.ShapeDtypeStruct((B * K,), jnp.float32),
        ],
        scratch_types=[
            pltpu.VMEM((BPW,), jnp.int32),
            pltpu.VMEM((BPW,), jnp.int32),
            pltpu.VMEM((BPW * K,), jnp.int32),
            pltpu.VMEM((C, D), jnp.float32),
            pltpu.VMEM((C, D), jnp.float32),
            pltpu.VMEM((C, D), jnp.float32),
            pltpu.VMEM((C, D), jnp.float32),
            pltpu.VMEM((C, D), jnp.float32),
            pltpu.VMEM((C, D), jnp.float32),
            pltpu.VMEM((C * K, D), jnp.float32),
            pltpu.VMEM((C * K, D), jnp.float32),
            pltpu.VMEM((C * K, D), jnp.float32),
            pltpu.VMEM((BPW,), jnp.float32),
            pltpu.VMEM((BPW * K,), jnp.float32),
            pltpu.SemaphoreType.DMA,
            pltpu.SemaphoreType.DMA,
            pltpu.SemaphoreType.DMA,
        ],
    )
    def k(pv_hbm, pu_hbm, ng_hbm, vw_hbm, uw_hbm, pos_out, neg_out,
          pv_idx, pu_idx, ng_idx, v_a, v_b, v_c, u_a, u_b, u_c,
          n_a, n_b, n_c, p_all, n_all, sem_a, sem_b, sem_c):
        wid = lax.axis_index("s") * NC + lax.axis_index("c")
        pltpu.sync_copy(pv_hbm.at[pl.ds(wid * BPW, BPW)], pv_idx)
        pltpu.sync_copy(pu_hbm.at[pl.ds(wid * BPW, BPW)], pu_idx)
        pltpu.sync_copy(ng_hbm.at[pl.ds(wid * BPW * K, BPW * K)], ng_idx)

        bufs = [(v_a, u_a, n_a, sem_a), (v_b, u_b, n_b, sem_b),
                (v_c, u_c, n_c, sem_c)]
        NB = len(bufs)
        lane = lax.broadcasted_iota(jnp.int32, (L,), 0)
        perms = [lane ^ (1 << s) for s in range(4)]

        def fire(g):
            v_r, u_r, n_r, sem = bufs[g % NB]
            cps = [
                pltpu.async_copy(
                    vw_hbm.at[pv_idx.at[pl.ds(g * C, C)]], v_r, sem),
                pltpu.async_copy(
                    uw_hbm.at[pu_idx.at[pl.ds(g * C, C)]], u_r, sem),
            ]
            for kk in range(K):
                cps.append(pltpu.async_copy(
                    uw_hbm.at[ng_idx.at[pl.ds(g * C * K + kk * C, C)]],
                    n_r.at[pl.ds(kk * C, C)], sem))
            return cps

        zeros = jnp.zeros((L,), jnp.float32)

        @plsc.parallel_loop(0, BPW // L)
        def zero_p(i):
            p_all[pl.ds(i * L, L)] = zeros

        @plsc.parallel_loop(0, BPW * K // L)
        def zero_n(i):
            n_all[pl.ds(i * L, L)] = zeros

        pend = {g: fire(g) for g in range(NB - 1)}
        for g in range(G):
            if g + NB - 1 < G:
                pend[g + NB - 1] = fire(g + NB - 1)
            for cp in pend.pop(g):
                cp.wait()
            v_r, u_r, n_r, _ = bufs[g % NB]

            @plsc.parallel_loop(0, C)
            def elem(i, v_r=v_r, u_r=u_r, n_r=n_r, g=g):
                slot = lane == (i % L)
                base = g * C + (i // L) * L
                vv = [v_r[i, pl.ds(L * j, L)] for j in range(NV)]

                def dot_total(row_ref, r):
                    acc = vv[0] * row_ref[r, pl.ds(0, L)]
                    for j in range(1, NV):
                        acc = acc + vv[j] * row_ref[r, pl.ds(L * j, L)]
                    for p in perms:
                        acc = acc + jnp.take_along_axis(acc, p, axis=0)
                    return acc

                plsc.addupdate(p_all.at[pl.ds(base, L)],
                               jnp.where(slot, dot_total(u_r, i), zeros))
                for kk in range(K):
                    plsc.addupdate(
                        n_all.at[pl.ds(kk * BPW + base, L)],
                        jnp.where(slot, dot_total(n_r, kk * C + i), zeros))

        pltpu.sync_copy(p_all, pos_out.at[pl.ds(wid * BPW, BPW)])
        pltpu.sync_copy(n_all, neg_out.at[pl.ds(wid * BPW * K, BPW * K)])

    return k(pos_v, pos_u, neg_u, v_weight, u_weight)


def _tc_reduce_body(pos_ref, neg_ref, out_ref):
    tot = jnp.sum(jax.nn.log_sigmoid(pos_ref[...]))
    tot = tot + jnp.sum(jax.nn.log_sigmoid(-neg_ref[...]))
    out_ref[0, 0] = -tot


def _tc_reduce(pos_s, neg_s):
    return pl.pallas_call(
        _tc_reduce_body,
        out_shape=jax.ShapeDtypeStruct((1, 1), jnp.float32),
        out_specs=pl.BlockSpec(memory_space=pltpu.SMEM),
    )(pos_s.reshape(B // D, D), neg_s.reshape(B * K // D, D))


def kernel(pos_v, pos_u, neg_u, v_weight, u_weight):
    pos_v = pos_v.astype(jnp.int32)
    pos_u = pos_u.astype(jnp.int32)
    neg_u = neg_u.astype(jnp.int32).reshape(B * K)
    pos_s, neg_s = _sc_scores(pos_v, pos_u, neg_u, v_weight, u_weight)
    out = _tc_reduce(pos_s, neg_s)
    return out[0, 0]


# speedup vs baseline: 1.0359x; 1.0359x over previous
import functools

import jax
import jax.numpy as jnp
from jax import lax
from jax.experimental import pallas as pl
from jax.experimental.pallas import tpu as pltpu
from jax.experimental.pallas import tpu_sc as plsc

V = 100000
D = 128
B = 16384
K = 5
L = 16
NC = 2
NS = 16
NW = NC * NS
BPW = B // NW
C = 64
G = BPW // C
NV = D // L


def _sc_scores(pos_v, pos_u, neg_u, v_weight, u_weight):
    mesh = plsc.VectorSubcoreMesh(core_axis_name="c", subcore_axis_name="s")

    @functools.partial(
        pl.kernel,
        mesh=mesh,
        out_type=[
            jax.ShapeDtypeStruct((B,), jnp.float32),
            jax.ShapeDtypeStruct((B * K,), jnp.float32),
        ],
        scratch_types=[
            pltpu.VMEM((BPW,), jnp.int32),
            pltpu.VMEM((BPW,), jnp.int32),
            pltpu.VMEM((BPW * K,), jnp.int32),
            pltpu.VMEM((C, D), jnp.float32),
            pltpu.VMEM((C, D), jnp.float32),
            pltpu.VMEM((C, D), jnp.float32),
            pltpu.VMEM((C, D), jnp.float32),
            pltpu.VMEM((C * K, D), jnp.float32),
            pltpu.VMEM((C * K, D), jnp.float32),
            pltpu.VMEM((BPW,), jnp.float32),
            pltpu.VMEM((BPW * K,), jnp.float32),
            pltpu.SemaphoreType.DMA,
            pltpu.SemaphoreType.DMA,
        ],
    )
    def k(pv_hbm, pu_hbm, ng_hbm, vw_hbm, uw_hbm, pos_out, neg_out,
          pv_idx, pu_idx, ng_idx, v_a, v_b, u_a, u_b,
          n_a, n_b, p_all, n_all, sem_a, sem_b):
        wid = lax.axis_index("s") * NC + lax.axis_index("c")
        pltpu.sync_copy(pv_hbm.at[pl.ds(wid * BPW, BPW)], pv_idx)
        pltpu.sync_copy(pu_hbm.at[pl.ds(wid * BPW, BPW)], pu_idx)
        pltpu.sync_copy(ng_hbm.at[pl.ds(wid * BPW * K, BPW * K)], ng_idx)

        bufs = [(v_a, u_a, n_a, sem_a), (v_b, u_b, n_b, sem_b)]
        NB = len(bufs)
        lane = lax.broadcasted_iota(jnp.int32, (L,), 0)
        perms = [lane ^ (1 << s) for s in range(4)]

        def fire(g):
            v_r, u_r, n_r, sem = bufs[g % NB]
            cps = [
                pltpu.async_copy(
                    vw_hbm.at[pv_idx.at[pl.ds(g * C, C)]], v_r, sem),
                pltpu.async_copy(
                    uw_hbm.at[pu_idx.at[pl.ds(g * C, C)]], u_r, sem),
            ]
            for kk in range(K):
                cps.append(pltpu.async_copy(
                    uw_hbm.at[ng_idx.at[pl.ds(g * C * K + kk * C, C)]],
                    n_r.at[pl.ds(kk * C, C)], sem))
            return cps

        zeros = jnp.zeros((L,), jnp.float32)

        @plsc.parallel_loop(0, BPW // L)
        def zero_p(i):
            p_all[pl.ds(i * L, L)] = zeros

        @plsc.parallel_loop(0, BPW * K // L)
        def zero_n(i):
            n_all[pl.ds(i * L, L)] = zeros

        pend = {g: fire(g) for g in range(NB - 1)}
        for g in range(G):
            if g + NB - 1 < G:
                pend[g + NB - 1] = fire(g + NB - 1)
            for cp in pend.pop(g):
                cp.wait()
            v_r, u_r, n_r, _ = bufs[g % NB]

            @plsc.parallel_loop(0, C)
            def elem(i, v_r=v_r, u_r=u_r, n_r=n_r, g=g):
                slot = lane == (i % L)
                base = g * C + (i // L) * L
                vv = [v_r[i, pl.ds(L * j, L)] for j in range(NV)]

                def dot_total(row_ref, r):
                    acc = vv[0] * row_ref[r, pl.ds(0, L)]
                    for j in range(1, NV):
                        acc = acc + vv[j] * row_ref[r, pl.ds(L * j, L)]
                    for p in perms:
                        acc = acc + jnp.take_along_axis(acc, p, axis=0)
                    return acc

                plsc.addupdate(p_all.at[pl.ds(base, L)],
                               jnp.where(slot, dot_total(u_r, i), zeros))
                for kk in range(K):
                    plsc.addupdate(
                        n_all.at[pl.ds(kk * BPW + base, L)],
                        jnp.where(slot, dot_total(n_r, kk * C + i), zeros))

        pltpu.sync_copy(p_all, pos_out.at[pl.ds(wid * BPW, BPW)])
        pltpu.sync_copy(n_all, neg_out.at[pl.ds(wid * BPW * K, BPW * K)])

    return k(pos_v, pos_u, neg_u, v_weight, u_weight)


def _tc_reduce_body(pos_ref, neg_ref, out_ref):
    tot = jnp.sum(jax.nn.log_sigmoid(pos_ref[...]))
    tot = tot + jnp.sum(jax.nn.log_sigmoid(-neg_ref[...]))
    out_ref[0, 0] = -tot


def _tc_reduce(pos_s, neg_s):
    return pl.pallas_call(
        _tc_reduce_body,
        out_shape=jax.ShapeDtypeStruct((1, 1), jnp.float32),
        out_specs=pl.BlockSpec(memory_space=pltpu.SMEM),
    )(pos_s.reshape(B // D, D), neg_s.reshape(B * K // D, D))


def kernel(pos_v, pos_u, neg_u, v_weight, u_weight):
    pos_v = pos_v.astype(jnp.int32)
    pos_u = pos_u.astype(jnp.int32)
    neg_u = neg_u.astype(jnp.int32).reshape(B * K)
    pos_s, neg_s = _sc_scores(pos_v, pos_u, neg_u, v_weight, u_weight)
    out = _tc_reduce(pos_s, neg_s)
    return out[0, 0]


# speedup vs baseline: 1.0369x; 1.0010x over previous
import functools

import jax
import jax.numpy as jnp
from jax import lax
from jax.experimental import pallas as pl
from jax.experimental.pallas import tpu as pltpu
from jax.experimental.pallas import tpu_sc as plsc

V = 100000
D = 128
B = 16384
K = 5
L = 16
NC = 2
NS = 16
NW = NC * NS
BPW = B // NW
C = 64
G = BPW // C
NV = D // L


def _sc_scores(pos_v, pos_u, neg_u, v_weight, u_weight):
    mesh = plsc.VectorSubcoreMesh(core_axis_name="c", subcore_axis_name="s")

    @functools.partial(
        pl.kernel,
        mesh=mesh,
        out_type=[
            jax.ShapeDtypeStruct((B,), jnp.float32),
            jax.ShapeDtypeStruct((B * K,), jnp.float32),
        ],
        scratch_types=[
            pltpu.VMEM((BPW,), jnp.int32),
            pltpu.VMEM((BPW,), jnp.int32),
            pltpu.VMEM((BPW * K,), jnp.int32),
            pltpu.VMEM((C, D), jnp.float32),
            pltpu.VMEM((C, D), jnp.float32),
            pltpu.VMEM((C, D), jnp.float32),
            pltpu.VMEM((C, D), jnp.float32),
            pltpu.VMEM((C * K, D), jnp.float32),
            pltpu.VMEM((C * K, D), jnp.float32),
            pltpu.VMEM((BPW,), jnp.float32),
            pltpu.VMEM((BPW * K,), jnp.float32),
            pltpu.SemaphoreType.DMA,
            pltpu.SemaphoreType.DMA,
        ],
    )
    def k(pv_hbm, pu_hbm, ng_hbm, vw_hbm, uw_hbm, pos_out, neg_out,
          pv_idx, pu_idx, ng_idx, v_a, v_b, u_a, u_b,
          n_a, n_b, p_all, n_all, sem_a, sem_b):
        wid = lax.axis_index("s") * NC + lax.axis_index("c")
        pltpu.sync_copy(pv_hbm.at[pl.ds(wid * BPW, BPW)], pv_idx)
        pltpu.sync_copy(pu_hbm.at[pl.ds(wid * BPW, BPW)], pu_idx)
        pltpu.sync_copy(ng_hbm.at[pl.ds(wid * BPW * K, BPW * K)], ng_idx)

        bufs = [(v_a, u_a, n_a, sem_a), (v_b, u_b, n_b, sem_b)]
        NB = len(bufs)
        lane = lax.broadcasted_iota(jnp.int32, (L,), 0)
        perms = [lane ^ (1 << s) for s in range(4)]

        def fire(g):
            v_r, u_r, n_r, sem = bufs[g % NB]
            cps = [
                pltpu.async_copy(
                    vw_hbm.at[pv_idx.at[pl.ds(g * C, C)]], v_r, sem),
                pltpu.async_copy(
                    uw_hbm.at[pu_idx.at[pl.ds(g * C, C)]], u_r, sem),
            ]
            for off in range(0, C * K, D):
                ln = min(D, C * K - off)
                cps.append(pltpu.async_copy(
                    uw_hbm.at[ng_idx.at[pl.ds(g * C * K + off, ln)]],
                    n_r.at[pl.ds(off, ln)], sem))
            return cps

        zeros = jnp.zeros((L,), jnp.float32)

        @plsc.parallel_loop(0, BPW // L)
        def zero_p(i):
            p_all[pl.ds(i * L, L)] = zeros

        @plsc.parallel_loop(0, BPW * K // L)
        def zero_n(i):
            n_all[pl.ds(i * L, L)] = zeros

        pend = {g: fire(g) for g in range(NB - 1)}
        for g in range(G):
            if g + NB - 1 < G:
                pend[g + NB - 1] = fire(g + NB - 1)
            for cp in pend.pop(g):
                cp.wait()
            v_r, u_r, n_r, _ = bufs[g % NB]

            @plsc.parallel_loop(0, C)
            def elem(i, v_r=v_r, u_r=u_r, n_r=n_r, g=g):
                slot = lane == (i % L)
                base = g * C + (i // L) * L
                vv = [v_r[i, pl.ds(L * j, L)] for j in range(NV)]

                def dot_total(row_ref, r):
                    acc = vv[0] * row_ref[r, pl.ds(0, L)]
                    for j in range(1, NV):
                        acc = acc + vv[j] * row_ref[r, pl.ds(L * j, L)]
                    for p in perms:
                        acc = acc + jnp.take_along_axis(acc, p, axis=0)
                    return acc

                plsc.addupdate(p_all.at[pl.ds(base, L)],
                               jnp.where(slot, dot_total(u_r, i), zeros))
                for kk in range(K):
                    plsc.addupdate(
                        n_all.at[pl.ds(kk * BPW + base, L)],
                        jnp.where(slot, dot_total(n_r, kk * C + i), zeros))

        pltpu.sync_copy(p_all, pos_out.at[pl.ds(wid * BPW, BPW)])
        pltpu.sync_copy(n_all, neg_out.at[pl.ds(wid * BPW * K, BPW * K)])

    return k(pos_v, pos_u, neg_u, v_weight, u_weight)


def _tc_reduce_body(pos_ref, neg_ref, out_ref):
    tot = jnp.sum(jax.nn.log_sigmoid(pos_ref[...]))
    tot = tot + jnp.sum(jax.nn.log_sigmoid(-neg_ref[...]))
    out_ref[0, 0] = -tot


def _tc_reduce(pos_s, neg_s):
    return pl.pallas_call(
        _tc_reduce_body,
        out_shape=jax.ShapeDtypeStruct((1, 1), jnp.float32),
        out_specs=pl.BlockSpec(memory_space=pltpu.SMEM),
    )(pos_s.reshape(B // D, D), neg_s.reshape(B * K // D, D))


def kernel(pos_v, pos_u, neg_u, v_weight, u_weight):
    pos_v = pos_v.astype(jnp.int32)
    pos_u = pos_u.astype(jnp.int32)
    neg_u = neg_u.astype(jnp.int32).reshape(B * K)
    pos_s, neg_s = _sc_scores(pos_v, pos_u, neg_u, v_weight, u_weight)
    out = _tc_reduce(pos_s, neg_s)
    return out[0, 0]
